# baseline (speedup 1.0000x reference)
import jax
import jax.numpy as jnp
from jax.experimental import pallas as pl
from jax.experimental.pallas import tpu as pltpu

B, S, T, M = 32, 512, 2048, 80
ROWS = B * T
BLK = 2048
GRID = ROWS // BLK


def _body(mt_ref, mp_ref, pmp_ref, mw_ref,
          pt_ref, pp_ref, et_ref, ep_ref,
          ldp_ref, dt_ref, pst_ref, psp_ref, sw_ref,
          out_ref, acc_ref):
    i = pl.program_id(0)

    @pl.when(i == 0)
    def _small():
        sw = sw_ref[...]
        n_src = jnp.sum(sw)
        s_pitch = jnp.sum((pp_ref[...] - pt_ref[...]) ** 2 * sw)
        s_energy = jnp.sum((ep_ref[...] - et_ref[...]) ** 2 * sw)
        ldt = jnp.log(dt_ref[...].astype(jnp.float32) + 1.0)
        s_dur = jnp.sum((ldp_ref[...] - ldt) ** 2 * sw)
        psp = psp_ref[...]
        pst = pst_ref[...]
        d = psp - pst
        s_mid = jnp.sum(d * d)
        cond = jnp.logical_and((0.0 * psp) > (psp - 0.5), pst != 0.0)
        s_pen = jnp.sum(cond.astype(jnp.float32))
        acc_ref[0] = 0.0
        acc_ref[1] = 0.0
        acc_ref[2] = 0.0
        acc_ref[3] = s_pitch
        acc_ref[4] = s_energy
        acc_ref[5] = s_dur
        acc_ref[6] = n_src
        acc_ref[7] = s_mid
        acc_ref[8] = s_pen

    w = mw_ref[...]
    mt = mt_ref[...]
    acc_ref[0] += jnp.sum(jnp.abs(mp_ref[...] - mt) * w)
    acc_ref[1] += jnp.sum(jnp.abs(pmp_ref[...] - mt) * w)
    acc_ref[2] += jnp.sum(w)

    @pl.when(i == GRID - 1)
    def _final():
        n_mel = acc_ref[2] * M
        mel_loss = acc_ref[0] / n_mel
        postnet_loss = acc_ref[1] / n_mel
        n_src = acc_ref[6]
        pitch_loss = acc_ref[3] / n_src
        energy_loss = acc_ref[4] / n_src
        dur_loss = acc_ref[5] / n_src
        pause_loss = (acc_ref[7] / (B * S) + 100.0 * 0.5 * acc_ref[8] / B) / S
        pause_w = pause_loss * 0.7
        out_ref[1] = mel_loss
        out_ref[2] = postnet_loss
        out_ref[3] = pitch_loss
        out_ref[4] = energy_loss
        out_ref[5] = dur_loss
        out_ref[6] = pause_w
        out_ref[0] = (mel_loss + postnet_loss + dur_loss + pitch_loss +
                      energy_loss + pause_w)


def kernel(mel_targets, pitch_targets, energy_targets, pause_targets,
           mel_predictions, postnet_mel_predictions, pitch_predictions,
           energy_predictions, log_duration_predictions, pause_predictions,
           duration_targets, src_masks, mel_masks):
    mt2 = mel_targets.reshape(ROWS, M)
    mp2 = mel_predictions.reshape(ROWS, M)
    pmp2 = postnet_mel_predictions.reshape(ROWS, M)
    mw = jnp.logical_not(mel_masks).astype(jnp.float32).reshape(ROWS, 1)
    sw = jnp.logical_not(src_masks).astype(jnp.float32).reshape(128, 128)

    def r2(x):
        return x.reshape(128, 128)

    mel_spec = pl.BlockSpec((BLK, M), lambda i: (i, 0))
    mw_spec = pl.BlockSpec((BLK, 1), lambda i: (i, 0))
    small_spec = pl.BlockSpec((128, 128), lambda i: (0, 0))

    out = pl.pallas_call(
        _body,
        grid=(GRID,),
        in_specs=[mel_spec, mel_spec, mel_spec, mw_spec] + [small_spec] * 9,
        out_specs=pl.BlockSpec(memory_space=pltpu.SMEM),
        out_shape=jax.ShapeDtypeStruct((8,), jnp.float32),
        scratch_shapes=[pltpu.SMEM((16,), jnp.float32)],
    )(mt2, mp2, pmp2, mw,
      r2(pitch_targets), r2(pitch_predictions),
      r2(energy_targets), r2(energy_predictions),
      r2(log_duration_predictions), r2(duration_targets),
      r2(pause_targets), r2(pause_predictions), sw)

    return (out[0], out[1], out[2], out[3], out[4], out[5], out[6])


# speedup vs baseline: 1.0433x; 1.0433x over previous
import jax
import jax.numpy as jnp
from jax.experimental import pallas as pl
from jax.experimental.pallas import tpu as pltpu

B, S, T, M = 32, 512, 2048, 80
ROWS = B * T
BLK = 2048
GRID = ROWS // BLK


def _body(mt_ref, mp_ref, pmp_ref, mw_ref, mwf_ref,
          pt_ref, pp_ref, et_ref, ep_ref,
          ldp_ref, dt_ref, pst_ref, psp_ref, sw_ref,
          out_ref, acc1_ref, acc2_ref, sacc_ref):
    i = pl.program_id(0)

    @pl.when(i == 0)
    def _small():
        sw = sw_ref[...]
        n_src = jnp.sum(sw)
        s_pitch = jnp.sum((pp_ref[...] - pt_ref[...]) ** 2 * sw)
        s_energy = jnp.sum((ep_ref[...] - et_ref[...]) ** 2 * sw)
        ldt = jnp.log(dt_ref[...].astype(jnp.float32) + 1.0)
        s_dur = jnp.sum((ldp_ref[...] - ldt) ** 2 * sw)
        psp = psp_ref[...]
        pst = pst_ref[...]
        d = psp - pst
        s_mid = jnp.sum(d * d)
        cond = jnp.logical_and((0.0 * psp) > (psp - 0.5), pst != 0.0)
        s_pen = jnp.sum(cond.astype(jnp.float32))
        sacc_ref[0] = s_pitch
        sacc_ref[1] = s_energy
        sacc_ref[2] = s_dur
        sacc_ref[3] = n_src
        sacc_ref[4] = s_mid
        sacc_ref[5] = s_pen
        sacc_ref[6] = jnp.sum(mwf_ref[...])
        acc1_ref[...] = jnp.zeros_like(acc1_ref)
        acc2_ref[...] = jnp.zeros_like(acc2_ref)

    w2 = mw_ref[0]
    mt = mt_ref[...]
    a1 = jnp.abs(mp_ref[...] - mt)
    a2 = jnp.abs(pmp_ref[...] - mt)
    acc1_ref[...] += jax.lax.dot_general(
        w2, a1, (((1,), (0,)), ((), ())),
        preferred_element_type=jnp.float32,
        precision=jax.lax.Precision.HIGHEST)
    acc2_ref[...] += jax.lax.dot_general(
        w2, a2, (((1,), (0,)), ((), ())),
        preferred_element_type=jnp.float32,
        precision=jax.lax.Precision.HIGHEST)

    @pl.when(i == GRID - 1)
    def _final():
        n_mel = sacc_ref[6] * M
        mel_loss = jnp.sum(acc1_ref[...]) / n_mel
        postnet_loss = jnp.sum(acc2_ref[...]) / n_mel
        n_src = sacc_ref[3]
        pitch_loss = sacc_ref[0] / n_src
        energy_loss = sacc_ref[1] / n_src
        dur_loss = sacc_ref[2] / n_src
        pause_loss = (sacc_ref[4] / (B * S) + 100.0 * 0.5 * sacc_ref[5] / B) / S
        pause_w = pause_loss * 0.7
        out_ref[1] = mel_loss
        out_ref[2] = postnet_loss
        out_ref[3] = pitch_loss
        out_ref[4] = energy_loss
        out_ref[5] = dur_loss
        out_ref[6] = pause_w
        out_ref[0] = (mel_loss + postnet_loss + dur_loss + pitch_loss +
                      energy_loss + pause_w)


def kernel(mel_targets, pitch_targets, energy_targets, pause_targets,
           mel_predictions, postnet_mel_predictions, pitch_predictions,
           energy_predictions, log_duration_predictions, pause_predictions,
           duration_targets, src_masks, mel_masks):
    mt2 = mel_targets.reshape(ROWS, M)
    mp2 = mel_predictions.reshape(ROWS, M)
    pmp2 = postnet_mel_predictions.reshape(ROWS, M)
    mwf = jnp.logical_not(mel_masks).astype(jnp.float32)
    mw = mwf.reshape(GRID, 1, BLK)
    sw = jnp.logical_not(src_masks).astype(jnp.float32).reshape(128, 128)

    def r2(x):
        return x.reshape(128, 128)

    mel_spec = pl.BlockSpec((BLK, M), lambda i: (i, 0))
    mw_spec = pl.BlockSpec((1, 1, BLK), lambda i: (i, 0, 0))
    small_spec = pl.BlockSpec((128, 128), lambda i: (0, 0))
    mwf_spec = pl.BlockSpec((512, 128), lambda i: (0, 0))

    out = pl.pallas_call(
        _body,
        grid=(GRID,),
        in_specs=[mel_spec, mel_spec, mel_spec, mw_spec, mwf_spec] +
                 [small_spec] * 9,
        out_specs=pl.BlockSpec(memory_space=pltpu.SMEM),
        out_shape=jax.ShapeDtypeStruct((8,), jnp.float32),
        scratch_shapes=[pltpu.VMEM((1, M), jnp.float32),
                        pltpu.VMEM((1, M), jnp.float32),
                        pltpu.SMEM((8,), jnp.float32)],
    )(mt2, mp2, pmp2, mw, mwf.reshape(512, 128),
      r2(pitch_targets), r2(pitch_predictions),
      r2(energy_targets), r2(energy_predictions),
      r2(log_duration_predictions), r2(duration_targets),
      r2(pause_targets), r2(pause_predictions), sw)

    return (out[0], out[1], out[2], out[3], out[4], out[5], out[6])
